# baseline (speedup 1.0000x reference)
import jax
import jax.numpy as jnp
from jax.experimental import pallas as pl
from jax.experimental.pallas import tpu as pltpu

_NCHUNK = 4


def _copy_body(total, nchunk):
    sz = total // nchunk

    def body(w_hbm, o_hbm, vmem, rsem, wsem):
        reads = []
        for i in range(nchunk):
            sl = pl.ds(i * sz, sz)
            reads.append(pltpu.make_async_copy(w_hbm.at[sl], vmem.at[i], rsem.at[i]))
        for r in reads:
            r.start()
        writes = []
        for i in range(nchunk):
            sl = pl.ds(i * sz, sz)
            reads[i].wait()
            w = pltpu.make_async_copy(vmem.at[i], o_hbm.at[sl], wsem.at[i])
            w.start()
            writes.append(w)
        for w in writes:
            w.wait()

    return body


def kernel(indices, weight):
    seq_len = indices.shape[-2]
    cols = weight.shape[1]
    flat = jnp.reshape(weight[:seq_len], (seq_len * cols,))
    nchunk = _NCHUNK
    while (seq_len * cols) % nchunk:
        nchunk //= 2
    out = pl.pallas_call(
        _copy_body(seq_len * cols, nchunk),
        out_shape=jax.ShapeDtypeStruct((seq_len * cols,), weight.dtype),
        in_specs=[pl.BlockSpec(memory_space=pl.ANY)],
        out_specs=pl.BlockSpec(memory_space=pl.ANY),
        scratch_shapes=[
            pltpu.VMEM((nchunk, (seq_len * cols) // nchunk), weight.dtype),
            pltpu.SemaphoreType.DMA((nchunk,)),
            pltpu.SemaphoreType.DMA((nchunk,)),
        ],
    )(flat)
    return jnp.reshape(out, (seq_len, cols))


# speedup vs baseline: 1.4914x; 1.4914x over previous
import jax
import jax.numpy as jnp
from jax.experimental import pallas as pl
from jax.experimental.pallas import tpu as pltpu

_NCHUNK = 4
_WIDE_COLS = 8192


def _copy_body(rows, nchunk):
    blk = rows // nchunk

    def body(w_hbm, o_hbm, vmem, rsem, wsem):
        reads = []
        for i in range(nchunk):
            sl = pl.ds(i * blk, blk)
            reads.append(pltpu.make_async_copy(w_hbm.at[sl, :], vmem.at[i], rsem.at[i]))
        for r in reads:
            r.start()
        writes = []
        for i in range(nchunk):
            sl = pl.ds(i * blk, blk)
            reads[i].wait()
            w = pltpu.make_async_copy(vmem.at[i], o_hbm.at[sl, :], wsem.at[i])
            w.start()
            writes.append(w)
        for w in writes:
            w.wait()

    return body


def kernel(indices, weight):
    seq_len = indices.shape[-2]
    cols = weight.shape[1]
    total = seq_len * cols
    wide = _WIDE_COLS if total % (_WIDE_COLS * _NCHUNK * 8) == 0 else cols
    rows = total // wide
    flat = jnp.reshape(weight[:seq_len], (rows, wide))
    out = pl.pallas_call(
        _copy_body(rows, _NCHUNK),
        out_shape=jax.ShapeDtypeStruct((rows, wide), weight.dtype),
        in_specs=[pl.BlockSpec(memory_space=pl.ANY)],
        out_specs=pl.BlockSpec(memory_space=pl.ANY),
        scratch_shapes=[
            pltpu.VMEM((_NCHUNK, rows // _NCHUNK, wide), weight.dtype),
            pltpu.SemaphoreType.DMA((_NCHUNK,)),
            pltpu.SemaphoreType.DMA((_NCHUNK,)),
        ],
    )(flat)
    return jnp.reshape(out, (seq_len, cols))


# speedup vs baseline: 6.8633x; 4.6020x over previous
import jax
import jax.numpy as jnp
from jax.experimental import pallas as pl
from jax.experimental.pallas import tpu as pltpu

_NCHUNK = 16


def _copy_body(seq_len, cols, nchunk):
    rows = seq_len // nchunk

    def body(w_hbm, o_hbm, vmem, rsem, wsem):
        reads = []
        for i in range(nchunk):
            sl = pl.ds(i * rows, rows)
            reads.append(pltpu.make_async_copy(w_hbm.at[sl, :], vmem.at[i], rsem.at[i]))
        for r in reads:
            r.start()
        writes = []
        for i in range(nchunk):
            sl = pl.ds(i * rows, rows)
            reads[i].wait()
            w = pltpu.make_async_copy(vmem.at[i], o_hbm.at[sl, :], wsem.at[i])
            w.start()
            writes.append(w)
        for w in writes:
            w.wait()

    return body


def kernel(indices, weight):
    seq_len = indices.shape[-2]
    cols = weight.shape[1]
    nchunk = _NCHUNK
    while seq_len % nchunk:
        nchunk //= 2
    rows = seq_len // nchunk
    return pl.pallas_call(
        _copy_body(seq_len, cols, nchunk),
        out_shape=jax.ShapeDtypeStruct((seq_len, cols), weight.dtype),
        in_specs=[pl.BlockSpec(memory_space=pl.ANY)],
        out_specs=pl.BlockSpec(memory_space=pl.ANY),
        scratch_shapes=[
            pltpu.VMEM((nchunk, rows, cols), weight.dtype),
            pltpu.SemaphoreType.DMA((nchunk,)),
            pltpu.SemaphoreType.DMA((nchunk,)),
        ],
    )(weight)


# speedup vs baseline: 7.1884x; 1.0474x over previous
import jax
import jax.numpy as jnp
from jax.experimental import pallas as pl
from jax.experimental.pallas import tpu as pltpu

_NCHUNK = 4


def _copy_body(seq_len, cols, nchunk):
    rows = seq_len // nchunk

    def body(w_hbm, o_hbm, vmem, rsem, wsem):
        reads = []
        for i in range(nchunk):
            sl = pl.ds(i * rows, rows)
            reads.append(pltpu.make_async_copy(w_hbm.at[sl, :], vmem.at[i], rsem.at[i]))
        for r in reads:
            r.start()
        writes = []
        for i in range(nchunk):
            sl = pl.ds(i * rows, rows)
            reads[i].wait()
            w = pltpu.make_async_copy(vmem.at[i], o_hbm.at[sl, :], wsem.at[i])
            w.start()
            writes.append(w)
        for w in writes:
            w.wait()

    return body


def kernel(indices, weight):
    seq_len = indices.shape[-2]
    cols = weight.shape[1]
    nchunk = _NCHUNK
    while seq_len % nchunk:
        nchunk //= 2
    rows = seq_len // nchunk
    return pl.pallas_call(
        _copy_body(seq_len, cols, nchunk),
        out_shape=jax.ShapeDtypeStruct((seq_len, cols), weight.dtype),
        in_specs=[pl.BlockSpec(memory_space=pl.ANY)],
        out_specs=pl.BlockSpec(memory_space=pl.ANY),
        scratch_shapes=[
            pltpu.VMEM((nchunk, rows, cols), weight.dtype),
            pltpu.SemaphoreType.DMA((nchunk,)),
            pltpu.SemaphoreType.DMA((nchunk,)),
        ],
    )(weight)


# speedup vs baseline: 7.2564x; 1.0095x over previous
import jax
import jax.numpy as jnp
from jax.experimental import pallas as pl
from jax.experimental.pallas import tpu as pltpu


def _chunk_rows(seq_len):
    if seq_len % 8 == 0:
        u = seq_len // 8
        return [u, 3 * u, 3 * u, u]
    return [seq_len]


def _copy_body(offsets, sizes):
    def body(w_hbm, o_hbm, *refs):
        n = len(sizes)
        vmems = refs[:n]
        rsem, wsem = refs[n], refs[n + 1]
        reads = []
        for i, (off, sz) in enumerate(zip(offsets, sizes)):
            sl = pl.ds(off, sz)
            reads.append(pltpu.make_async_copy(w_hbm.at[sl, :], vmems[i], rsem.at[i]))
        for r in reads:
            r.start()
        writes = []
        for i, (off, sz) in enumerate(zip(offsets, sizes)):
            sl = pl.ds(off, sz)
            reads[i].wait()
            w = pltpu.make_async_copy(vmems[i], o_hbm.at[sl, :], wsem.at[i])
            w.start()
            writes.append(w)
        for w in writes:
            w.wait()

    return body


def kernel(indices, weight):
    seq_len = indices.shape[-2]
    cols = weight.shape[1]
    sizes = _chunk_rows(seq_len)
    offsets = [sum(sizes[:i]) for i in range(len(sizes))]
    n = len(sizes)
    return pl.pallas_call(
        _copy_body(offsets, sizes),
        out_shape=jax.ShapeDtypeStruct((seq_len, cols), weight.dtype),
        in_specs=[pl.BlockSpec(memory_space=pl.ANY)],
        out_specs=pl.BlockSpec(memory_space=pl.ANY),
        scratch_shapes=(
            [pltpu.VMEM((sz, cols), weight.dtype) for sz in sizes]
            + [pltpu.SemaphoreType.DMA((n,)), pltpu.SemaphoreType.DMA((n,))]
        ),
    )(weight)


# speedup vs baseline: 7.4173x; 1.0222x over previous
import jax
import jax.numpy as jnp
from jax.experimental import pallas as pl
from jax.experimental.pallas import tpu as pltpu


def _chunk_rows(seq_len):
    if seq_len % 16 == 0:
        u = seq_len // 16
        return [u, 2 * u, 5 * u, 5 * u, 2 * u, u]
    return [seq_len]


def _copy_body(offsets, sizes):
    def body(w_hbm, o_hbm, *refs):
        n = len(sizes)
        vmems = refs[:n]
        rsem, wsem = refs[n], refs[n + 1]
        reads = []
        for i, (off, sz) in enumerate(zip(offsets, sizes)):
            sl = pl.ds(off, sz)
            reads.append(pltpu.make_async_copy(w_hbm.at[sl, :], vmems[i], rsem.at[i]))
        for r in reads:
            r.start()
        writes = []
        for i, (off, sz) in enumerate(zip(offsets, sizes)):
            sl = pl.ds(off, sz)
            reads[i].wait()
            w = pltpu.make_async_copy(vmems[i], o_hbm.at[sl, :], wsem.at[i])
            w.start()
            writes.append(w)
        for w in writes:
            w.wait()

    return body


def kernel(indices, weight):
    seq_len = indices.shape[-2]
    cols = weight.shape[1]
    sizes = _chunk_rows(seq_len)
    offsets = [sum(sizes[:i]) for i in range(len(sizes))]
    n = len(sizes)
    return pl.pallas_call(
        _copy_body(offsets, sizes),
        out_shape=jax.ShapeDtypeStruct((seq_len, cols), weight.dtype),
        in_specs=[pl.BlockSpec(memory_space=pl.ANY)],
        out_specs=pl.BlockSpec(memory_space=pl.ANY),
        scratch_shapes=(
            [pltpu.VMEM((sz, cols), weight.dtype) for sz in sizes]
            + [pltpu.SemaphoreType.DMA((n,)), pltpu.SemaphoreType.DMA((n,))]
        ),
    )(weight)
